# Initial kernel scaffold; baseline (speedup 1.0000x reference)
import jax
import jax.numpy as jnp
from jax.experimental import pallas as pl


def kernel(hidden_states, attention_mask, layer_head_mask, Wq, bq, Wk, bk, Wv, bv, Wo, bo, ln_gamma, ln_beta, is_index_masked, is_index_global_attn, is_global_attn):
    raise NotImplementedError("write your pallas kernel here")


# speedup vs baseline: 1.0959x; 1.0959x over previous
import functools
import math

import jax
import jax.numpy as jnp
from jax.experimental import pallas as pl

B, S, D, H = 1, 2048, 2048, 16
HD = D // H
WIN = 256
HALF = WIN // 2
LN_EPS = 1e-5

RB = 256
QB = 256
NQ = S // QB


def _qkv_kernel(hs_ref, w_ref, b_ref, out_ref):
    acc = jnp.dot(hs_ref[...], w_ref[...], preferred_element_type=jnp.float32)
    out_ref[...] = acc + b_ref[...]


def _attn_kernel(q_ref, k0_ref, k1_ref, k2_ref, v0_ref, v1_ref, v2_ref,
                 am0_ref, am1_ref, am2_ref, rowmul_ref, hm_ref, out_ref):
    h = pl.program_id(0)
    i = pl.program_id(1)
    del h
    q = q_ref[...]
    k = jnp.concatenate([k0_ref[...], k1_ref[...], k2_ref[...]], axis=0)
    v = jnp.concatenate([v0_ref[...], v1_ref[...], v2_ref[...]], axis=0)
    am = jnp.concatenate([am0_ref[...], am1_ref[...], am2_ref[...]], axis=1)

    scores = jax.lax.dot_general(
        q, k, (((1,), (1,)), ((), ())), preferred_element_type=jnp.float32
    )
    scores = scores + am

    big = jnp.int32(1 << 20)
    base0 = jnp.where(i > 0, -QB, big)
    base2 = jnp.where(i < NQ - 1, QB, big)
    qrow = jax.lax.broadcasted_iota(jnp.int32, (QB, 3 * QB), 0)
    off = jax.lax.broadcasted_iota(jnp.int32, (QB, 3 * QB), 1)
    krel = off % QB + jnp.where(off < QB, base0, jnp.where(off < 2 * QB, 0, base2))
    diff = jnp.abs(qrow - krel)
    scores = jnp.where(diff <= HALF, scores, -1e9)

    m = jnp.max(scores, axis=-1, keepdims=True)
    e = jnp.exp(scores - m)
    probs = e / jnp.sum(e, axis=-1, keepdims=True)

    ctx = jnp.dot(probs, v, preferred_element_type=jnp.float32)
    ctx = ctx * hm_ref[0, 0, 0]
    ctx = ctx * rowmul_ref[0, :].reshape(QB, 1)
    out_ref[...] = ctx


def _out_kernel(ctx_ref, wo_ref, bo_ref, hs_ref, g_ref, bta_ref, out_ref):
    o = jnp.dot(ctx_ref[...], wo_ref[...], preferred_element_type=jnp.float32)
    y = o + bo_ref[...] + hs_ref[...]
    mu = jnp.mean(y, axis=-1, keepdims=True)
    yc = y - mu
    var = jnp.mean(yc * yc, axis=-1, keepdims=True)
    y = yc * jax.lax.rsqrt(var + LN_EPS)
    out_ref[...] = y * g_ref[...] + bta_ref[...]


def kernel(hidden_states, attention_mask, layer_head_mask, Wq, bq, Wk, bk, Wv, bv,
           Wo, bo, ln_gamma, ln_beta, is_index_masked, is_index_global_attn,
           is_global_attn):
    hs = hidden_states.reshape(S, D)
    inv = 1.0 / math.sqrt(HD)
    wqkv = jnp.concatenate([Wq * inv, Wk, Wv], axis=1)
    bqkv = jnp.concatenate([bq * inv, bk, bv]).reshape(1, 3 * D)

    CB = 1024
    qkv = pl.pallas_call(
        _qkv_kernel,
        grid=(3 * D // CB, S // RB),
        in_specs=[
            pl.BlockSpec((RB, D), lambda j, i: (i, 0)),
            pl.BlockSpec((D, CB), lambda j, i: (0, j)),
            pl.BlockSpec((1, CB), lambda j, i: (0, j)),
        ],
        out_specs=pl.BlockSpec((RB, CB), lambda j, i: (i, j)),
        out_shape=jax.ShapeDtypeStruct((S, 3 * D), jnp.float32),
    )(hs, wqkv, bqkv)

    q = qkv[:, :D]
    k = qkv[:, D:2 * D]
    v = qkv[:, 2 * D:]

    am = attention_mask.reshape(1, S)
    rowmul = (1.0 - is_index_masked.astype(jnp.float32)).reshape(1, S)
    hm = layer_head_mask.reshape(H, 1, 1)

    qspec = pl.BlockSpec((QB, HD), lambda h, i: (i, h))
    k_prev = pl.BlockSpec((QB, HD), lambda h, i: (jnp.maximum(i - 1, 0), h))
    k_self = pl.BlockSpec((QB, HD), lambda h, i: (i, h))
    k_next = pl.BlockSpec((QB, HD), lambda h, i: (jnp.minimum(i + 1, NQ - 1), h))
    am_prev = pl.BlockSpec((1, QB), lambda h, i: (0, jnp.maximum(i - 1, 0)))
    am_self = pl.BlockSpec((1, QB), lambda h, i: (0, i))
    am_next = pl.BlockSpec((1, QB), lambda h, i: (0, jnp.minimum(i + 1, NQ - 1)))

    ctx = pl.pallas_call(
        _attn_kernel,
        grid=(H, NQ),
        in_specs=[
            qspec, k_prev, k_self, k_next, k_prev, k_self, k_next,
            am_prev, am_self, am_next,
            pl.BlockSpec((1, QB), lambda h, i: (0, i)),
            pl.BlockSpec((1, 1, 1), lambda h, i: (h, 0, 0)),
        ],
        out_specs=pl.BlockSpec((QB, HD), lambda h, i: (i, h)),
        out_shape=jax.ShapeDtypeStruct((S, D), jnp.float32),
    )(q, k, k, k, v, v, v, am, am, am, rowmul, hm)

    y = pl.pallas_call(
        _out_kernel,
        grid=(S // RB,),
        in_specs=[
            pl.BlockSpec((RB, D), lambda i: (i, 0)),
            pl.BlockSpec((D, D), lambda i: (0, 0)),
            pl.BlockSpec((1, D), lambda i: (0, 0)),
            pl.BlockSpec((RB, D), lambda i: (i, 0)),
            pl.BlockSpec((1, D), lambda i: (0, 0)),
            pl.BlockSpec((1, D), lambda i: (0, 0)),
        ],
        out_specs=pl.BlockSpec((RB, D), lambda i: (i, 0)),
        out_shape=jax.ShapeDtypeStruct((S, D), jnp.float32),
    )(ctx, Wo, bo.reshape(1, D), hs, ln_gamma.reshape(1, D), ln_beta.reshape(1, D))

    return y.reshape(B, S, D)


# speedup vs baseline: 1.2257x; 1.1184x over previous
import functools
import math

import jax
import jax.numpy as jnp
from jax.experimental import pallas as pl

B, S, D, H = 1, 2048, 2048, 16
HD = D // H
WIN = 256
HALF = WIN // 2
LN_EPS = 1e-5

RB = 256
QB = 256
NQ = S // QB


def _qkv_kernel(hs_ref, w_ref, b_ref, out_ref):
    acc = jnp.dot(hs_ref[...], w_ref[...], preferred_element_type=jnp.float32)
    out_ref[...] = (acc + b_ref[...]).astype(jnp.bfloat16)


def _attn_kernel(q_ref, k0_ref, k1_ref, k2_ref, v0_ref, v1_ref, v2_ref,
                 am0_ref, am1_ref, am2_ref, rowmul_ref, hm_ref, out_ref):
    h = pl.program_id(0)
    i = pl.program_id(1)
    del h
    q = q_ref[...]
    k = jnp.concatenate([k0_ref[...], k1_ref[...], k2_ref[...]], axis=0)
    v = jnp.concatenate([v0_ref[...], v1_ref[...], v2_ref[...]], axis=0)
    am = jnp.concatenate([am0_ref[...], am1_ref[...], am2_ref[...]], axis=1)

    scores = jax.lax.dot_general(
        q, k, (((1,), (1,)), ((), ())), preferred_element_type=jnp.float32
    )
    scores = scores + am

    big = jnp.int32(1 << 20)
    base0 = jnp.where(i > 0, -QB, big)
    base2 = jnp.where(i < NQ - 1, QB, big)
    qrow = jax.lax.broadcasted_iota(jnp.int32, (QB, 3 * QB), 0)
    off = jax.lax.broadcasted_iota(jnp.int32, (QB, 3 * QB), 1)
    krel = off % QB + jnp.where(off < QB, base0, jnp.where(off < 2 * QB, 0, base2))
    diff = jnp.abs(qrow - krel)
    scores = jnp.where(diff <= HALF, scores, -1e9)

    m = jnp.max(scores, axis=-1, keepdims=True)
    e = jnp.exp(scores - m)
    probs = e / jnp.sum(e, axis=-1, keepdims=True)

    ctx = jnp.dot(probs.astype(jnp.bfloat16), v,
                  preferred_element_type=jnp.float32)
    ctx = ctx * hm_ref[0, 0, 0]
    ctx = ctx * rowmul_ref[0, :].reshape(QB, 1)
    out_ref[...] = ctx.astype(jnp.bfloat16)


def _out_kernel(ctx_ref, wo_ref, bo_ref, hs_ref, g_ref, bta_ref, out_ref):
    o = jnp.dot(ctx_ref[...], wo_ref[...], preferred_element_type=jnp.float32)
    y = o + bo_ref[...] + hs_ref[...]
    mu = jnp.mean(y, axis=-1, keepdims=True)
    yc = y - mu
    var = jnp.mean(yc * yc, axis=-1, keepdims=True)
    y = yc * jax.lax.rsqrt(var + LN_EPS)
    out_ref[...] = y * g_ref[...] + bta_ref[...]


def kernel(hidden_states, attention_mask, layer_head_mask, Wq, bq, Wk, bk, Wv, bv,
           Wo, bo, ln_gamma, ln_beta, is_index_masked, is_index_global_attn,
           is_global_attn):
    hs = hidden_states.reshape(S, D)
    inv = 1.0 / math.sqrt(HD)
    wqkv = jnp.concatenate([Wq * inv, Wk, Wv], axis=1).astype(jnp.bfloat16)
    bqkv = jnp.concatenate([bq * inv, bk, bv]).reshape(1, 3 * D)
    hs_bf = hs.astype(jnp.bfloat16)

    CB = 1024
    qkv = pl.pallas_call(
        _qkv_kernel,
        grid=(3 * D // CB, S // RB),
        in_specs=[
            pl.BlockSpec((RB, D), lambda j, i: (i, 0)),
            pl.BlockSpec((D, CB), lambda j, i: (0, j)),
            pl.BlockSpec((1, CB), lambda j, i: (0, j)),
        ],
        out_specs=pl.BlockSpec((RB, CB), lambda j, i: (i, j)),
        out_shape=jax.ShapeDtypeStruct((S, 3 * D), jnp.bfloat16),
    )(hs_bf, wqkv, bqkv)

    q = qkv[:, :D]
    k = qkv[:, D:2 * D]
    v = qkv[:, 2 * D:]

    am = attention_mask.reshape(1, S)
    rowmul = (1.0 - is_index_masked.astype(jnp.float32)).reshape(1, S)
    hm = layer_head_mask.reshape(H, 1, 1)

    qspec = pl.BlockSpec((QB, HD), lambda h, i: (i, h))
    k_prev = pl.BlockSpec((QB, HD), lambda h, i: (jnp.maximum(i - 1, 0), h))
    k_self = pl.BlockSpec((QB, HD), lambda h, i: (i, h))
    k_next = pl.BlockSpec((QB, HD), lambda h, i: (jnp.minimum(i + 1, NQ - 1), h))
    am_prev = pl.BlockSpec((1, QB), lambda h, i: (0, jnp.maximum(i - 1, 0)))
    am_self = pl.BlockSpec((1, QB), lambda h, i: (0, i))
    am_next = pl.BlockSpec((1, QB), lambda h, i: (0, jnp.minimum(i + 1, NQ - 1)))

    ctx = pl.pallas_call(
        _attn_kernel,
        grid=(H, NQ),
        in_specs=[
            qspec, k_prev, k_self, k_next, k_prev, k_self, k_next,
            am_prev, am_self, am_next,
            pl.BlockSpec((1, QB), lambda h, i: (0, i)),
            pl.BlockSpec((1, 1, 1), lambda h, i: (h, 0, 0)),
        ],
        out_specs=pl.BlockSpec((QB, HD), lambda h, i: (i, h)),
        out_shape=jax.ShapeDtypeStruct((S, D), jnp.bfloat16),
    )(q, k, k, k, v, v, v, am, am, am, rowmul, hm)

    y = pl.pallas_call(
        _out_kernel,
        grid=(S // RB,),
        in_specs=[
            pl.BlockSpec((RB, D), lambda i: (i, 0)),
            pl.BlockSpec((D, D), lambda i: (0, 0)),
            pl.BlockSpec((1, D), lambda i: (0, 0)),
            pl.BlockSpec((RB, D), lambda i: (i, 0)),
            pl.BlockSpec((1, D), lambda i: (0, 0)),
            pl.BlockSpec((1, D), lambda i: (0, 0)),
        ],
        out_specs=pl.BlockSpec((RB, D), lambda i: (i, 0)),
        out_shape=jax.ShapeDtypeStruct((S, D), jnp.float32),
    )(ctx, Wo.astype(jnp.bfloat16), bo.reshape(1, D), hs,
      ln_gamma.reshape(1, D), ln_beta.reshape(1, D))

    return y.reshape(B, S, D)


# speedup vs baseline: 1.7911x; 1.4613x over previous
import math

import jax
import jax.numpy as jnp
import numpy as np
from jax.experimental import pallas as pl

B, S, D, H = 1, 2048, 2048, 16
HD = D // H
WIN = 256
HALF = WIN // 2
LN_EPS = 1e-5

RB = 256
QB = 256
NQ = S // QB
KBS = 128
NKB = S // KBS
SPAN = 4 * KBS

_r = np.arange(QB)[:, None]
_c = np.arange(SPAN)[None, :]
_band = np.abs(_r - (_c - KBS)) <= HALF
_pen_int = np.where(_band, 0.0, -1e9).astype(np.float32)
_pen_first = _pen_int.copy()
_pen_first[:, :KBS] = -1e9
_pen_last = _pen_int.copy()
_pen_last[:, 3 * KBS:] = -1e9
_PEN3 = np.stack([_pen_first, _pen_int, _pen_last])


def _qkv_kernel(hs_ref, w_ref, b_ref, out_ref):
    acc = jnp.dot(hs_ref[...], w_ref[...], preferred_element_type=jnp.float32)
    out_ref[...] = (acc + b_ref[...]).astype(jnp.bfloat16)


def _attn_out_kernel(q_ref, k0_ref, k1_ref, k2_ref, k3_ref,
                     v0_ref, v1_ref, v2_ref, v3_ref,
                     am0_ref, am1_ref, am2_ref, am3_ref,
                     pen_ref, rowmul_ref, hs_ref, wo_ref, bo_ref,
                     g_ref, bta_ref, out_ref):
    am = jnp.concatenate(
        [am0_ref[...], am1_ref[...], am2_ref[...], am3_ref[...]], axis=1)
    pen = pen_ref[0] + am
    rowv = rowmul_ref[0, :].reshape(QB, 1)
    krefs = (k0_ref, k1_ref, k2_ref, k3_ref)
    vrefs = (v0_ref, v1_ref, v2_ref, v3_ref)

    ctx_parts = []
    for h in range(H):
        sl = slice(h * HD, (h + 1) * HD)
        qh = q_ref[:, sl]
        s = jnp.concatenate(
            [jax.lax.dot_general(qh, kr[:, sl], (((1,), (1,)), ((), ())),
                                 preferred_element_type=jnp.float32)
             for kr in krefs], axis=1)
        s = s + pen
        m = jnp.max(s, axis=-1, keepdims=True)
        e = jnp.exp(s - m)
        l = jnp.sum(e, axis=-1, keepdims=True)
        eb = e.astype(jnp.bfloat16)
        acc = jnp.dot(eb[:, :KBS], vrefs[0][:, sl],
                      preferred_element_type=jnp.float32)
        for j in range(1, 4):
            acc = acc + jnp.dot(eb[:, j * KBS:(j + 1) * KBS], vrefs[j][:, sl],
                                preferred_element_type=jnp.float32)
        ctx_parts.append((acc * (rowv / l)).astype(jnp.bfloat16))

    ctx = jnp.concatenate(ctx_parts, axis=1)
    o = jnp.dot(ctx, wo_ref[...], preferred_element_type=jnp.float32)
    y = o + bo_ref[...] + hs_ref[...]
    mu = jnp.mean(y, axis=-1, keepdims=True)
    yc = y - mu
    var = jnp.mean(yc * yc, axis=-1, keepdims=True)
    y = yc * jax.lax.rsqrt(var + LN_EPS)
    out_ref[...] = y * g_ref[...] + bta_ref[...]


def kernel(hidden_states, attention_mask, layer_head_mask, Wq, bq, Wk, bk, Wv, bv,
           Wo, bo, ln_gamma, ln_beta, is_index_masked, is_index_global_attn,
           is_global_attn):
    hs = hidden_states.reshape(S, D)
    inv = 1.0 / math.sqrt(HD)
    wqkv = jnp.concatenate([Wq * inv, Wk, Wv], axis=1).astype(jnp.bfloat16)
    bqkv = jnp.concatenate([bq * inv, bk, bv]).reshape(1, 3 * D)
    hs_bf = hs.astype(jnp.bfloat16)

    CB = 1024
    qkv = pl.pallas_call(
        _qkv_kernel,
        grid=(3 * D // CB, S // RB),
        in_specs=[
            pl.BlockSpec((RB, D), lambda j, i: (i, 0)),
            pl.BlockSpec((D, CB), lambda j, i: (0, j)),
            pl.BlockSpec((1, CB), lambda j, i: (0, j)),
        ],
        out_specs=pl.BlockSpec((RB, CB), lambda j, i: (i, j)),
        out_shape=jax.ShapeDtypeStruct((S, 3 * D), jnp.bfloat16),
    )(hs_bf, wqkv, bqkv)

    q = qkv[:, :D]
    k = qkv[:, D:2 * D]
    v = qkv[:, 2 * D:]

    am = attention_mask.reshape(1, S)
    rowmul = (1.0 - is_index_masked.astype(jnp.float32)).reshape(1, S)
    wo_scaled = (Wo * jnp.repeat(layer_head_mask, HD)[:, None]).astype(jnp.bfloat16)
    pen3 = jnp.asarray(_PEN3)

    k0 = pl.BlockSpec((KBS, D), lambda i: (jnp.maximum(2 * i - 1, 0), 0))
    k1 = pl.BlockSpec((KBS, D), lambda i: (2 * i, 0))
    k2 = pl.BlockSpec((KBS, D), lambda i: (2 * i + 1, 0))
    k3 = pl.BlockSpec((KBS, D), lambda i: (jnp.minimum(2 * i + 2, NKB - 1), 0))
    a0 = pl.BlockSpec((1, KBS), lambda i: (0, jnp.maximum(2 * i - 1, 0)))
    a1 = pl.BlockSpec((1, KBS), lambda i: (0, 2 * i))
    a2 = pl.BlockSpec((1, KBS), lambda i: (0, 2 * i + 1))
    a3 = pl.BlockSpec((1, KBS), lambda i: (0, jnp.minimum(2 * i + 2, NKB - 1)))
    pen_spec = pl.BlockSpec(
        (1, QB, SPAN),
        lambda i: (jnp.where(i == 0, 0, jnp.where(i == NQ - 1, 2, 1)), 0, 0))

    y = pl.pallas_call(
        _attn_out_kernel,
        grid=(NQ,),
        in_specs=[
            pl.BlockSpec((QB, D), lambda i: (i, 0)),
            k0, k1, k2, k3, k0, k1, k2, k3,
            a0, a1, a2, a3,
            pen_spec,
            pl.BlockSpec((1, QB), lambda i: (0, i)),
            pl.BlockSpec((QB, D), lambda i: (i, 0)),
            pl.BlockSpec((D, D), lambda i: (0, 0)),
            pl.BlockSpec((1, D), lambda i: (0, 0)),
            pl.BlockSpec((1, D), lambda i: (0, 0)),
            pl.BlockSpec((1, D), lambda i: (0, 0)),
        ],
        out_specs=pl.BlockSpec((QB, D), lambda i: (i, 0)),
        out_shape=jax.ShapeDtypeStruct((S, D), jnp.float32),
    )(q, k, k, k, k, v, v, v, v, am, am, am, am, pen3, rowmul, hs,
      wo_scaled, bo.reshape(1, D), ln_gamma.reshape(1, D), ln_beta.reshape(1, D))

    return y.reshape(B, S, D)


# speedup vs baseline: 1.9821x; 1.1066x over previous
import math

import jax
import jax.numpy as jnp
import numpy as np
from jax.experimental import pallas as pl

B, S, D, H = 1, 2048, 2048, 16
HD = D // H
WIN = 256
HALF = WIN // 2
LN_EPS = 1e-5

RB = 256
QB = 256
NQ = S // QB
KBS = 128
NKB = S // KBS
SPAN = 4 * KBS

_r = np.arange(QB)[:, None]
_c = np.arange(SPAN)[None, :]
_band = np.abs(_r - (_c - KBS)) <= HALF
_pen_int = np.where(_band, 0.0, -1e9).astype(np.float32)
_pen_first = _pen_int.copy()
_pen_first[:, :KBS] = -1e9
_pen_last = _pen_int.copy()
_pen_last[:, 3 * KBS:] = -1e9
_PEN3 = np.stack([_pen_first, _pen_int, _pen_last])


def _qkv_kernel(hs_ref, w_ref, b_ref, out_ref):
    acc = jnp.dot(hs_ref[...], w_ref[...], preferred_element_type=jnp.float32)
    out_ref[...] = (acc + b_ref[...]).astype(jnp.bfloat16)


def _attn_out_kernel(q_ref, k0_ref, k1_ref, k2_ref, k3_ref,
                     v0_ref, v1_ref, v2_ref, v3_ref,
                     am0_ref, am1_ref, am2_ref, am3_ref,
                     pen_ref, rowmul_ref, hs_ref, wo_ref, bo_ref,
                     g_ref, bta_ref, out_ref):
    am = jnp.concatenate(
        [am0_ref[...], am1_ref[...], am2_ref[...], am3_ref[...]], axis=1)
    pen = pen_ref[0] + am
    rowv = rowmul_ref[0, :].reshape(QB, 1)
    krefs = (k0_ref, k1_ref, k2_ref, k3_ref)
    vrefs = (v0_ref, v1_ref, v2_ref, v3_ref)

    ctx_parts = []
    for h in range(H):
        sl = slice(h * HD, (h + 1) * HD)
        qh = q_ref[:, sl]
        s = jnp.concatenate(
            [jax.lax.dot_general(qh, kr[:, sl], (((1,), (1,)), ((), ())),
                                 preferred_element_type=jnp.float32)
             for kr in krefs], axis=1)
        e = jnp.exp(jnp.minimum(s + pen, 60.0))
        l = jnp.sum(e, axis=-1, keepdims=True)
        eb = e.astype(jnp.bfloat16)
        acc = jnp.dot(eb[:, :KBS], vrefs[0][:, sl],
                      preferred_element_type=jnp.float32)
        for j in range(1, 4):
            acc = acc + jnp.dot(eb[:, j * KBS:(j + 1) * KBS], vrefs[j][:, sl],
                                preferred_element_type=jnp.float32)
        ctx_parts.append((acc * (rowv / l)).astype(jnp.bfloat16))

    ctx = jnp.concatenate(ctx_parts, axis=1)
    o = jnp.dot(ctx, wo_ref[...], preferred_element_type=jnp.float32)
    y = o + bo_ref[...] + hs_ref[...]
    mu = jnp.mean(y, axis=-1, keepdims=True)
    yc = y - mu
    var = jnp.mean(yc * yc, axis=-1, keepdims=True)
    y = yc * jax.lax.rsqrt(var + LN_EPS)
    out_ref[...] = y * g_ref[...] + bta_ref[...]


def kernel(hidden_states, attention_mask, layer_head_mask, Wq, bq, Wk, bk, Wv, bv,
           Wo, bo, ln_gamma, ln_beta, is_index_masked, is_index_global_attn,
           is_global_attn):
    hs = hidden_states.reshape(S, D)
    inv = 1.0 / math.sqrt(HD)
    wqkv = jnp.concatenate([Wq * inv, Wk, Wv], axis=1).astype(jnp.bfloat16)
    bqkv = jnp.concatenate([bq * inv, bk, bv]).reshape(1, 3 * D)
    hs_bf = hs.astype(jnp.bfloat16)

    qkv = pl.pallas_call(
        _qkv_kernel,
        grid=(S // RB,),
        in_specs=[
            pl.BlockSpec((RB, D), lambda i: (i, 0)),
            pl.BlockSpec((D, 3 * D), lambda i: (0, 0)),
            pl.BlockSpec((1, 3 * D), lambda i: (0, 0)),
        ],
        out_specs=pl.BlockSpec((RB, 3 * D), lambda i: (i, 0)),
        out_shape=jax.ShapeDtypeStruct((S, 3 * D), jnp.bfloat16),
    )(hs_bf, wqkv, bqkv)

    q = qkv[:, :D]
    k = qkv[:, D:2 * D]
    v = qkv[:, 2 * D:]

    am = attention_mask.reshape(1, S)
    rowmul = (1.0 - is_index_masked.astype(jnp.float32)).reshape(1, S)
    wo_scaled = (Wo * jnp.repeat(layer_head_mask, HD)[:, None]).astype(jnp.bfloat16)
    pen3 = jnp.asarray(_PEN3)

    k0 = pl.BlockSpec((KBS, D), lambda i: (jnp.maximum(2 * i - 1, 0), 0))
    k1 = pl.BlockSpec((KBS, D), lambda i: (2 * i, 0))
    k2 = pl.BlockSpec((KBS, D), lambda i: (2 * i + 1, 0))
    k3 = pl.BlockSpec((KBS, D), lambda i: (jnp.minimum(2 * i + 2, NKB - 1), 0))
    a0 = pl.BlockSpec((1, KBS), lambda i: (0, jnp.maximum(2 * i - 1, 0)))
    a1 = pl.BlockSpec((1, KBS), lambda i: (0, 2 * i))
    a2 = pl.BlockSpec((1, KBS), lambda i: (0, 2 * i + 1))
    a3 = pl.BlockSpec((1, KBS), lambda i: (0, jnp.minimum(2 * i + 2, NKB - 1)))
    pen_spec = pl.BlockSpec(
        (1, QB, SPAN),
        lambda i: (jnp.where(i == 0, 0, jnp.where(i == NQ - 1, 2, 1)), 0, 0))

    y = pl.pallas_call(
        _attn_out_kernel,
        grid=(NQ,),
        in_specs=[
            pl.BlockSpec((QB, D), lambda i: (i, 0)),
            k0, k1, k2, k3, k0, k1, k2, k3,
            a0, a1, a2, a3,
            pen_spec,
            pl.BlockSpec((1, QB), lambda i: (0, i)),
            pl.BlockSpec((QB, D), lambda i: (i, 0)),
            pl.BlockSpec((D, D), lambda i: (0, 0)),
            pl.BlockSpec((1, D), lambda i: (0, 0)),
            pl.BlockSpec((1, D), lambda i: (0, 0)),
            pl.BlockSpec((1, D), lambda i: (0, 0)),
        ],
        out_specs=pl.BlockSpec((QB, D), lambda i: (i, 0)),
        out_shape=jax.ShapeDtypeStruct((S, D), jnp.float32),
    )(q, k, k, k, k, v, v, v, v, am, am, am, am, pen3, rowmul, hs,
      wo_scaled, bo.reshape(1, D), ln_gamma.reshape(1, D), ln_beta.reshape(1, D))

    return y.reshape(B, S, D)


# speedup vs baseline: 2.6440x; 1.3339x over previous
import math

import jax
import jax.numpy as jnp
import numpy as np
from jax.experimental import pallas as pl
from jax.experimental.pallas import tpu as pltpu

B, S, D, H = 1, 2048, 2048, 16
HD = D // H
WIN = 256
HALF = WIN // 2
LN_EPS = 1e-5

RB = 256
QB = 256
NQ = S // QB
KBS = 128
NKB = S // KBS
SPAN = 4 * KBS

_r = np.arange(QB)[:, None]
_c = np.arange(SPAN)[None, :]
_band = np.abs(_r - (_c - KBS)) <= HALF
_pen_int = np.where(_band, 0.0, -1e9).astype(np.float32)
_pen_first = _pen_int.copy()
_pen_first[:, :KBS] = -1e9
_pen_last = _pen_int.copy()
_pen_last[:, 3 * KBS:] = -1e9
_PEN3 = np.stack([_pen_first, _pen_int, _pen_last])


def _proj_kernel(hs_ref, w_ref, scale_ref, b_ref, out_ref, w_bf):
    i = pl.program_id(0)

    @pl.when(i == 0)
    def _():
        w_bf[...] = (w_ref[...] * scale_ref[...]).astype(jnp.bfloat16)

    hsb = hs_ref[...].astype(jnp.bfloat16)
    acc = jnp.dot(hsb, w_bf[...], preferred_element_type=jnp.float32)
    out_ref[...] = (acc + b_ref[...]).astype(jnp.bfloat16)


def _attn_out_kernel(q_ref, k0_ref, k1_ref, k2_ref, k3_ref,
                     v0_ref, v1_ref, v2_ref, v3_ref,
                     am0_ref, am1_ref, am2_ref, am3_ref,
                     pen_ref, rowmul_ref, hs_ref, wo_ref, bo_ref,
                     g_ref, bta_ref, out_ref, wo_bf):
    i = pl.program_id(0)

    @pl.when(i == 0)
    def _():
        wo_bf[...] = wo_ref[...].astype(jnp.bfloat16)

    am = jnp.concatenate(
        [am0_ref[...], am1_ref[...], am2_ref[...], am3_ref[...]], axis=1)
    pen = pen_ref[0] + am
    rowv = rowmul_ref[0, :].reshape(QB, 1)
    krefs = (k0_ref, k1_ref, k2_ref, k3_ref)
    vrefs = (v0_ref, v1_ref, v2_ref, v3_ref)

    ctx_parts = []
    for h in range(H):
        sl = slice(h * HD, (h + 1) * HD)
        qh = q_ref[:, sl]
        s = jnp.concatenate(
            [jax.lax.dot_general(qh, kr[:, sl], (((1,), (1,)), ((), ())),
                                 preferred_element_type=jnp.float32)
             for kr in krefs], axis=1)
        e = jnp.exp(jnp.minimum(s + pen, 60.0))
        l = jnp.sum(e, axis=-1, keepdims=True)
        eb = e.astype(jnp.bfloat16)
        acc = jnp.dot(eb[:, :KBS], vrefs[0][:, sl],
                      preferred_element_type=jnp.float32)
        for j in range(1, 4):
            acc = acc + jnp.dot(eb[:, j * KBS:(j + 1) * KBS], vrefs[j][:, sl],
                                preferred_element_type=jnp.float32)
        ctx_parts.append((acc * (rowv / l)).astype(jnp.bfloat16))

    ctx = jnp.concatenate(ctx_parts, axis=1)
    o = jnp.dot(ctx, wo_bf[...], preferred_element_type=jnp.float32)
    y = o + bo_ref[...] + hs_ref[...]
    mu = jnp.mean(y, axis=-1, keepdims=True)
    yc = y - mu
    var = jnp.mean(yc * yc, axis=-1, keepdims=True)
    y = yc * jax.lax.rsqrt(var + LN_EPS)
    out_ref[...] = y * g_ref[...] + bta_ref[...]


def _proj(hs, w, scale, b):
    return pl.pallas_call(
        _proj_kernel,
        grid=(S // RB,),
        in_specs=[
            pl.BlockSpec((RB, D), lambda i: (i, 0)),
            pl.BlockSpec((D, D), lambda i: (0, 0)),
            pl.BlockSpec((1, D), lambda i: (0, 0)),
            pl.BlockSpec((1, D), lambda i: (0, 0)),
        ],
        out_specs=pl.BlockSpec((RB, D), lambda i: (i, 0)),
        out_shape=jax.ShapeDtypeStruct((S, D), jnp.bfloat16),
        scratch_shapes=[pltpu.VMEM((D, D), jnp.bfloat16)],
    )(hs, w, scale.reshape(1, D), b.reshape(1, D))


def kernel(hidden_states, attention_mask, layer_head_mask, Wq, bq, Wk, bk, Wv, bv,
           Wo, bo, ln_gamma, ln_beta, is_index_masked, is_index_global_attn,
           is_global_attn):
    hs = hidden_states.reshape(S, D)
    inv = 1.0 / math.sqrt(HD)
    ones = jnp.ones((D,), jnp.float32)
    hm_cols = jnp.repeat(layer_head_mask, HD)

    q = _proj(hs, Wq, ones * inv, bq * inv)
    k = _proj(hs, Wk, ones, bk)
    v = _proj(hs, Wv, hm_cols, bv * hm_cols)

    am = attention_mask.reshape(1, S)
    rowmul = (1.0 - is_index_masked.astype(jnp.float32)).reshape(1, S)
    pen3 = jnp.asarray(_PEN3)

    k0 = pl.BlockSpec((KBS, D), lambda i: (jnp.maximum(2 * i - 1, 0), 0))
    k1 = pl.BlockSpec((KBS, D), lambda i: (2 * i, 0))
    k2 = pl.BlockSpec((KBS, D), lambda i: (2 * i + 1, 0))
    k3 = pl.BlockSpec((KBS, D), lambda i: (jnp.minimum(2 * i + 2, NKB - 1), 0))
    a0 = pl.BlockSpec((1, KBS), lambda i: (0, jnp.maximum(2 * i - 1, 0)))
    a1 = pl.BlockSpec((1, KBS), lambda i: (0, 2 * i))
    a2 = pl.BlockSpec((1, KBS), lambda i: (0, 2 * i + 1))
    a3 = pl.BlockSpec((1, KBS), lambda i: (0, jnp.minimum(2 * i + 2, NKB - 1)))
    pen_spec = pl.BlockSpec(
        (1, QB, SPAN),
        lambda i: (jnp.where(i == 0, 0, jnp.where(i == NQ - 1, 2, 1)), 0, 0))

    y = pl.pallas_call(
        _attn_out_kernel,
        grid=(NQ,),
        in_specs=[
            pl.BlockSpec((QB, D), lambda i: (i, 0)),
            k0, k1, k2, k3, k0, k1, k2, k3,
            a0, a1, a2, a3,
            pen_spec,
            pl.BlockSpec((1, QB), lambda i: (0, i)),
            pl.BlockSpec((QB, D), lambda i: (i, 0)),
            pl.BlockSpec((D, D), lambda i: (0, 0)),
            pl.BlockSpec((1, D), lambda i: (0, 0)),
            pl.BlockSpec((1, D), lambda i: (0, 0)),
            pl.BlockSpec((1, D), lambda i: (0, 0)),
        ],
        out_specs=pl.BlockSpec((QB, D), lambda i: (i, 0)),
        out_shape=jax.ShapeDtypeStruct((S, D), jnp.float32),
        scratch_shapes=[pltpu.VMEM((D, D), jnp.bfloat16)],
    )(q, k, k, k, k, v, v, v, v, am, am, am, am, pen3, rowmul, hs,
      Wo, bo.reshape(1, D), ln_gamma.reshape(1, D), ln_beta.reshape(1, D))

    return y.reshape(B, S, D)
